# baseline (speedup 1.0000x reference)
import jax
import jax.numpy as jnp
from jax import lax
from jax.experimental import pallas as pl
from jax.experimental.pallas import tpu as pltpu

_R = 256
_NBUF = 4


def _manual_kernel(n_chunks, x_hbm, o_hbm, inb, outb, isem, osem):
    for c in range(2):
        pltpu.make_async_copy(
            x_hbm.at[pl.ds(c * _R, _R)], inb.at[c], isem.at[c]).start()

    def outer(i, carry):
        c0 = i * _NBUF
        for b in range(_NBUF):
            cc = c0 + b
            b2 = (b + 2) % _NBUF
            pltpu.make_async_copy(
                x_hbm.at[pl.ds(0, _R)], inb.at[b], isem.at[b]).wait()

            @pl.when(cc >= 2)
            def _():
                pltpu.make_async_copy(
                    outb.at[b2], o_hbm.at[pl.ds(0, _R)], osem.at[b2]).wait()

            @pl.when(cc + 2 < n_chunks)
            def _():
                pltpu.make_async_copy(
                    x_hbm.at[pl.ds((cc + 2) * _R, _R)],
                    inb.at[b2], isem.at[b2]).start()

            v = inb[b]
            outb[b] = jnp.where(v > -1.0, jnp.float32(3.0), v + 1.0)
            pltpu.make_async_copy(
                outb.at[b], o_hbm.at[pl.ds(cc * _R, _R)], osem.at[b]).start()
        return carry

    lax.fori_loop(0, n_chunks // _NBUF, outer, 0)

    for cc in (n_chunks - 2, n_chunks - 1):
        b = cc % _NBUF
        pltpu.make_async_copy(
            outb.at[b], o_hbm.at[pl.ds(0, _R)], osem.at[b]).wait()


def kernel(x):
    b, m, n = x.shape
    rows = b * m
    n_chunks = rows // _R
    x2 = x.reshape(rows, n)
    out = pl.pallas_call(
        lambda *refs: _manual_kernel(n_chunks, *refs),
        in_specs=[pl.BlockSpec(memory_space=pl.ANY)],
        out_specs=pl.BlockSpec(memory_space=pl.ANY),
        out_shape=jax.ShapeDtypeStruct((rows, n), x.dtype),
        scratch_shapes=[
            pltpu.VMEM((_NBUF, _R, n), jnp.float32),
            pltpu.VMEM((_NBUF, _R, n), jnp.float32),
            pltpu.SemaphoreType.DMA((_NBUF,)),
            pltpu.SemaphoreType.DMA((_NBUF,)),
        ],
    )(x2)
    return out.reshape(b, m, n)


# speedup vs baseline: 1.0176x; 1.0176x over previous
import jax
import jax.numpy as jnp
from jax.experimental import pallas as pl


_BLOCK_ROWS = 880


def _ew_kernel(x_ref, o_ref):
    x = x_ref[...]
    o_ref[...] = jnp.where(x > -1.0, jnp.float32(3.0), x + 1.0)


def kernel(x):
    b, m, n = x.shape
    x2 = x.reshape(b * m, n)
    rows = b * m
    out = pl.pallas_call(
        _ew_kernel,
        grid=(pl.cdiv(rows, _BLOCK_ROWS),),
        in_specs=[pl.BlockSpec((_BLOCK_ROWS, n), lambda i: (i, 0))],
        out_specs=pl.BlockSpec((_BLOCK_ROWS, n), lambda i: (i, 0)),
        out_shape=jax.ShapeDtypeStruct((rows, n), x.dtype),
    )(x2)
    return out.reshape(b, m, n)
